# initial kernel scaffold (unmeasured)
import jax
import jax.numpy as jnp
from jax import lax
from jax.experimental import pallas as pl
from jax.experimental.pallas import tpu as pltpu

T_PER = 4096
D = 2048
F = 4096
E_LOCAL = 4

BT = 512
BF = 2048


def _peer_coords():
    return (1 - lax.axis_index("x"), lax.axis_index("y"), lax.axis_index("z"))


def _neighbor_barrier(peer):
    barrier_sem = pltpu.get_barrier_semaphore()
    pl.semaphore_signal(
        barrier_sem, inc=1, device_id=peer,
        device_id_type=pl.DeviceIdType.MESH,
    )
    pl.semaphore_wait(barrier_sem, 1)


def _exchange_tokens(x_bf, a2d):

    def body(x_ref, a_ref, t_ref, pa_ref, sx, rx, sa, ra):
        peer = _peer_coords()
        _neighbor_barrier(peer)

        t_ref[pl.ds(0, T_PER), :] = x_ref[...]

        rdma_x = pltpu.make_async_remote_copy(
            src_ref=x_ref,
            dst_ref=t_ref.at[pl.ds(T_PER, T_PER), :],
            send_sem=sx, recv_sem=rx,
            device_id=peer, device_id_type=pl.DeviceIdType.MESH,
        )
        rdma_a = pltpu.make_async_remote_copy(
            src_ref=a_ref, dst_ref=pa_ref,
            send_sem=sa, recv_sem=ra,
            device_id=peer, device_id_type=pl.DeviceIdType.MESH,
        )
        rdma_x.start()
        rdma_a.start()
        rdma_x.wait()
        rdma_a.wait()

    return pl.pallas_call(
        body,
        out_shape=(
            jax.ShapeDtypeStruct((2 * T_PER, D), jnp.bfloat16),
            jax.ShapeDtypeStruct(a2d.shape, jnp.int32),
        ),
        in_specs=[
            pl.BlockSpec(memory_space=pltpu.VMEM),
            pl.BlockSpec(memory_space=pltpu.VMEM),
        ],
        out_specs=(
            pl.BlockSpec(memory_space=pltpu.VMEM),
            pl.BlockSpec(memory_space=pltpu.VMEM),
        ),
        scratch_shapes=[pltpu.SemaphoreType.DMA] * 4,
        compiler_params=pltpu.CompilerParams(collective_id=0),
    )(x_bf, a2d)


def _moe_ffn(tokens, masks, w1, w2):
    n_tb = (2 * T_PER) // BT
    n_f = F // BF

    def body(t_ref, m_ref, w1_ref, w2_ref, p_ref):
        e = pl.program_id(1)
        f = pl.program_id(2)

        h = jnp.dot(t_ref[...], w1_ref[0], preferred_element_type=jnp.float32)
        h = jnp.maximum(h, 0.0).astype(jnp.bfloat16)
        p = jnp.dot(h, w2_ref[0], preferred_element_type=jnp.float32)
        contrib = (p * m_ref[...].astype(jnp.float32)).astype(jnp.bfloat16)

        @pl.when(jnp.logical_and(e == 0, f == 0))
        def _():
            p_ref[...] = contrib

        @pl.when(jnp.logical_or(e > 0, f > 0))
        def _():
            p_ref[...] += contrib

    return pl.pallas_call(
        body,
        grid=(n_tb, E_LOCAL, n_f),
        in_specs=[
            pl.BlockSpec((BT, D), lambda tb, e, f: (tb, 0)),
            pl.BlockSpec((BT, 1), lambda tb, e, f: (tb, e)),
            pl.BlockSpec((1, D, BF), lambda tb, e, f: (e, 0, f)),
            pl.BlockSpec((1, BF, D), lambda tb, e, f: (e, f, 0)),
        ],
        out_specs=pl.BlockSpec((BT, D), lambda tb, e, f: (tb, 0)),
        out_shape=jax.ShapeDtypeStruct((2 * T_PER, D), jnp.bfloat16),
    )(tokens, masks, w1, w2)


def _exchange_partials(peer_partial):

    def body(pp_ref, r_ref, send_sem, recv_sem):
        peer = _peer_coords()
        _neighbor_barrier(peer)

        rdma = pltpu.make_async_remote_copy(
            src_ref=pp_ref, dst_ref=r_ref,
            send_sem=send_sem, recv_sem=recv_sem,
            device_id=peer, device_id_type=pl.DeviceIdType.MESH,
        )
        rdma.start()
        rdma.wait()

    return pl.pallas_call(
        body,
        out_shape=jax.ShapeDtypeStruct((T_PER, D), jnp.bfloat16),
        in_specs=[pl.BlockSpec(memory_space=pltpu.VMEM)],
        out_specs=pl.BlockSpec(memory_space=pltpu.VMEM),
        scratch_shapes=[pltpu.SemaphoreType.DMA] * 2,
        compiler_params=pltpu.CompilerParams(collective_id=1),
    )(peer_partial)


def kernel(x, assign, W1, W2):
    my_x = lax.axis_index("x")

    x_bf = x.astype(jnp.bfloat16)
    w1 = W1.astype(jnp.bfloat16)
    w2 = W2.astype(jnp.bfloat16)
    a2d = assign.reshape(32, 128)

    tokens, peer_a2d = _exchange_tokens(x_bf, a2d)

    all_assign = jnp.concatenate([assign, peer_a2d.reshape(-1)])
    local_ids = jnp.arange(E_LOCAL, dtype=jnp.int32) + E_LOCAL * my_x
    masks = (all_assign[:, None] == local_ids[None, :]).astype(jnp.bfloat16)

    partials = _moe_ffn(tokens, masks, w1, w2)

    recv_partial = _exchange_partials(partials[T_PER:])
    return partials[:T_PER].astype(jnp.float32) + recv_partial.astype(
        jnp.float32
    )


# baseline (device time: 1785264 ns/iter reference)
import jax
import jax.numpy as jnp
from jax import lax
from jax.experimental import pallas as pl
from jax.experimental.pallas import tpu as pltpu

T_PER = 4096
D = 2048
F = 4096
E_LOCAL = 4

BT = 512
BF = 2048

_VMEM_LIMIT = 60 * 1024 * 1024


def _peer_coords():
    return (1 - lax.axis_index("x"), lax.axis_index("y"), lax.axis_index("z"))


def _neighbor_barrier(peer):
    barrier_sem = pltpu.get_barrier_semaphore()
    pl.semaphore_signal(
        barrier_sem, inc=1, device_id=peer,
        device_id_type=pl.DeviceIdType.MESH,
    )
    pl.semaphore_wait(barrier_sem, 1)


def _exchange_tokens(x_bf, a2d):

    def body(x_ref, a_ref, t_ref, pa_ref, sx, rx, sa, ra):
        peer = _peer_coords()
        _neighbor_barrier(peer)

        t_ref[pl.ds(0, T_PER), :] = x_ref[...]

        rdma_x = pltpu.make_async_remote_copy(
            src_ref=x_ref,
            dst_ref=t_ref.at[pl.ds(T_PER, T_PER), :],
            send_sem=sx, recv_sem=rx,
            device_id=peer, device_id_type=pl.DeviceIdType.MESH,
        )
        rdma_a = pltpu.make_async_remote_copy(
            src_ref=a_ref, dst_ref=pa_ref,
            send_sem=sa, recv_sem=ra,
            device_id=peer, device_id_type=pl.DeviceIdType.MESH,
        )
        rdma_x.start()
        rdma_a.start()
        rdma_x.wait()
        rdma_a.wait()

    return pl.pallas_call(
        body,
        out_shape=(
            jax.ShapeDtypeStruct((2 * T_PER, D), jnp.bfloat16),
            jax.ShapeDtypeStruct(a2d.shape, jnp.int32),
        ),
        in_specs=[
            pl.BlockSpec(memory_space=pltpu.VMEM),
            pl.BlockSpec(memory_space=pltpu.VMEM),
        ],
        out_specs=(
            pl.BlockSpec(memory_space=pltpu.VMEM),
            pl.BlockSpec(memory_space=pltpu.VMEM),
        ),
        scratch_shapes=[pltpu.SemaphoreType.DMA] * 4,
        compiler_params=pltpu.CompilerParams(
            collective_id=0, vmem_limit_bytes=_VMEM_LIMIT
        ),
    )(x_bf, a2d)


def _moe_ffn(tokens, masks, w1, w2):
    n_tb = (2 * T_PER) // BT
    n_f = F // BF

    def body(t_ref, m_ref, w1_ref, w2_ref, p_ref):
        e = pl.program_id(1)
        f = pl.program_id(2)

        h = jnp.dot(t_ref[...], w1_ref[0], preferred_element_type=jnp.float32)
        h = jnp.maximum(h, 0.0).astype(jnp.bfloat16)
        p = jnp.dot(h, w2_ref[0], preferred_element_type=jnp.float32)
        onehot = (
            lax.broadcasted_iota(jnp.int32, (1, E_LOCAL), 1) == e
        ).astype(jnp.float32)
        m = jnp.sum(
            m_ref[...].astype(jnp.float32) * onehot, axis=1, keepdims=True
        )
        contrib = (p * m).astype(jnp.bfloat16)

        @pl.when(jnp.logical_and(e == 0, f == 0))
        def _():
            p_ref[...] = contrib

        @pl.when(jnp.logical_or(e > 0, f > 0))
        def _():
            p_ref[...] += contrib

    return pl.pallas_call(
        body,
        grid=(n_tb, E_LOCAL, n_f),
        in_specs=[
            pl.BlockSpec((BT, D), lambda tb, e, f: (tb, 0)),
            pl.BlockSpec((BT, E_LOCAL), lambda tb, e, f: (tb, 0)),
            pl.BlockSpec((1, D, BF), lambda tb, e, f: (e, 0, f)),
            pl.BlockSpec((1, BF, D), lambda tb, e, f: (e, f, 0)),
        ],
        out_specs=pl.BlockSpec((BT, D), lambda tb, e, f: (tb, 0)),
        out_shape=jax.ShapeDtypeStruct((2 * T_PER, D), jnp.bfloat16),
        compiler_params=pltpu.CompilerParams(vmem_limit_bytes=_VMEM_LIMIT),
    )(tokens, masks, w1, w2)


def _exchange_partials(peer_partial):

    def body(pp_ref, r_ref, send_sem, recv_sem):
        peer = _peer_coords()
        _neighbor_barrier(peer)

        rdma = pltpu.make_async_remote_copy(
            src_ref=pp_ref, dst_ref=r_ref,
            send_sem=send_sem, recv_sem=recv_sem,
            device_id=peer, device_id_type=pl.DeviceIdType.MESH,
        )
        rdma.start()
        rdma.wait()

    return pl.pallas_call(
        body,
        out_shape=jax.ShapeDtypeStruct((T_PER, D), jnp.bfloat16),
        in_specs=[pl.BlockSpec(memory_space=pltpu.VMEM)],
        out_specs=pl.BlockSpec(memory_space=pltpu.VMEM),
        scratch_shapes=[pltpu.SemaphoreType.DMA] * 2,
        compiler_params=pltpu.CompilerParams(
            collective_id=1, vmem_limit_bytes=_VMEM_LIMIT
        ),
    )(peer_partial)


def kernel(x, assign, W1, W2):
    my_x = lax.axis_index("x")

    x_bf = x.astype(jnp.bfloat16)
    w1 = W1.astype(jnp.bfloat16)
    w2 = W2.astype(jnp.bfloat16)
    a2d = assign.reshape(32, 128)

    tokens, peer_a2d = _exchange_tokens(x_bf, a2d)

    all_assign = jnp.concatenate([assign, peer_a2d.reshape(-1)])
    local_ids = jnp.arange(E_LOCAL, dtype=jnp.int32) + E_LOCAL * my_x
    masks = (all_assign[:, None] == local_ids[None, :]).astype(jnp.bfloat16)

    partials = _moe_ffn(tokens, masks, w1, w2)

    recv_partial = _exchange_partials(partials[T_PER:])
    return partials[:T_PER].astype(jnp.float32) + recv_partial.astype(
        jnp.float32
    )
